# Initial kernel scaffold; baseline (speedup 1.0000x reference)
import jax
import jax.numpy as jnp
from jax.experimental import pallas as pl


def kernel(user_embeddings, subreddit_embeddings, batch_users, batch_subreddits, total_user_embeddings, total_subreddit_embeddings, negative_indices):
    raise NotImplementedError("write your pallas kernel here")


# speedup vs baseline: 1.3507x; 1.3507x over previous
import functools

import jax
import jax.numpy as jnp
from jax import lax
from jax.experimental import pallas as pl
from jax.experimental.pallas import tpu as pltpu
from jax.experimental.pallas import tpu_sc as plsc

B = 4096
D = 128
NEG = 5
EPS = 1e-8


def _sc_neg_stats(batch_users, neg_idx_t, total_user, total_sub):
    info = plsc.get_sparse_core_info()
    NC, NS, L = info.num_cores, info.num_subcores, info.num_lanes
    NW = NC * NS
    bpw = B // NW
    groups = bpw // L

    mesh = plsc.VectorSubcoreMesh(core_axis_name="c", subcore_axis_name="s")

    @functools.partial(
        pl.kernel,
        mesh=mesh,
        compiler_params=pltpu.CompilerParams(needs_layout_passes=False),
        out_type=[
            jax.ShapeDtypeStruct((NEG, B), jnp.float32),
            jax.ShapeDtypeStruct((NEG, B), jnp.float32),
            jax.ShapeDtypeStruct((B,), jnp.float32),
        ],
        scratch_types=[
            pltpu.VMEM((bpw,), jnp.int32),
            pltpu.VMEM((NEG, bpw), jnp.int32),
            pltpu.VMEM((bpw, D), jnp.float32),
            pltpu.VMEM((NEG, bpw, D), jnp.float32),
            pltpu.VMEM((NEG, bpw), jnp.float32),
            pltpu.VMEM((NEG, bpw), jnp.float32),
            pltpu.VMEM((bpw,), jnp.float32),
            pltpu.SemaphoreType.DMA,
        ],
    )
    def k(bu_hbm, nidx_hbm, tu_hbm, ts_hbm, dot_out, nv2_out, nu2_out,
          uidx_v, nidx_v, u_rows, v_rows, rdot, rnv2, rnu2, sem):
        wid = lax.axis_index("s") * NC + lax.axis_index("c")
        base = wid * bpw

        pltpu.sync_copy(bu_hbm.at[pl.ds(base, bpw)], uidx_v)
        pltpu.sync_copy(nidx_hbm.at[:, pl.ds(base, bpw)], nidx_v)
        cps = [pltpu.async_copy(tu_hbm.at[uidx_v], u_rows, sem)]
        for j in range(NEG):
            cps.append(
                pltpu.async_copy(ts_hbm.at[nidx_v.at[j]], v_rows.at[j], sem))
        for cp in cps:
            cp.wait()

        for g in range(groups):
            ridx = g * L + lax.iota(jnp.int32, L)
            zero = jnp.zeros((L,), jnp.float32)
            init = (zero,) * (1 + 2 * NEG)

            def body(d, carry, ridx=ridx):
                dd = jnp.full((L,), d, jnp.int32)
                u = plsc.load_gather(u_rows, [ridx, dd])
                outs = [carry[0] + u * u]
                for j in range(NEG):
                    jj = jnp.full((L,), j, jnp.int32)
                    v = plsc.load_gather(v_rows, [jj, ridx, dd])
                    outs.append(carry[1 + 2 * j] + u * v)
                    outs.append(carry[2 + 2 * j] + v * v)
                return tuple(outs)

            res = lax.fori_loop(0, D, body, init)
            rnu2[pl.ds(g * L, L)] = res[0]
            for j in range(NEG):
                rdot[j, pl.ds(g * L, L)] = res[1 + 2 * j]
                rnv2[j, pl.ds(g * L, L)] = res[2 + 2 * j]

        pltpu.sync_copy(rnu2, nu2_out.at[pl.ds(base, bpw)])
        pltpu.sync_copy(rdot, dot_out.at[:, pl.ds(base, bpw)])
        pltpu.sync_copy(rnv2, nv2_out.at[:, pl.ds(base, bpw)])

    return k(batch_users, neg_idx_t, total_user, total_sub)


def _tc_finish(ue, se, nd, nv2, nu2):
    R = B // 128

    def body(ue_ref, se_ref, nd_ref, nv2_ref, nu2_ref, out_ref):
        u = ue_ref[...]
        s = se_ref[...]
        dot = jnp.sum(u * s, axis=1)
        na = jnp.sqrt(jnp.sum(u * u, axis=1))
        nb = jnp.sqrt(jnp.sum(s * s, axis=1))
        pos_sum = jnp.sum(dot / jnp.maximum(na * nb, EPS))

        nu = jnp.sqrt(nu2_ref[...])
        neg_sum = jnp.float32(0.0)
        for j in range(NEG):
            ndj = nd_ref[j * R:(j + 1) * R, :]
            nvj = jnp.sqrt(nv2_ref[j * R:(j + 1) * R, :])
            neg_sum = neg_sum + jnp.sum(ndj / jnp.maximum(nu * nvj, EPS))

        out_ref[0, 0] = 1.0 - pos_sum / B + neg_sum / (NEG * B)

    return pl.pallas_call(
        body,
        out_shape=jax.ShapeDtypeStruct((1, 1), jnp.float32),
        out_specs=pl.BlockSpec(memory_space=pltpu.SMEM),
    )(ue, se, nd, nv2, nu2)


def kernel(user_embeddings, subreddit_embeddings, batch_users,
           batch_subreddits, total_user_embeddings, total_subreddit_embeddings,
           negative_indices):
    del batch_subreddits
    bu = batch_users.astype(jnp.int32)
    nidx_t = negative_indices.astype(jnp.int32).T
    nd, nv2, nu2 = _sc_neg_stats(bu, nidx_t, total_user_embeddings,
                                 total_subreddit_embeddings)
    loss = _tc_finish(user_embeddings, subreddit_embeddings,
                      nd.reshape(NEG * B // 128, 128),
                      nv2.reshape(NEG * B // 128, 128),
                      nu2.reshape(B // 128, 128))
    return loss[0, 0]


# speedup vs baseline: 2.7717x; 2.0522x over previous
import functools

import jax
import jax.numpy as jnp
from jax import lax
from jax.experimental import pallas as pl
from jax.experimental.pallas import tpu as pltpu
from jax.experimental.pallas import tpu_sc as plsc

B = 4096
D = 128
NEG = 5
EPS = 1e-8


def _sc_neg_stats(batch_users, neg_idx_flat, total_user, total_sub):
    info = plsc.get_sparse_core_info()
    NC, NS, L = info.num_cores, info.num_subcores, info.num_lanes
    NW = NC * NS
    bpw = B // NW
    vpw = bpw * NEG
    K = D // L

    mesh = plsc.VectorSubcoreMesh(core_axis_name="c", subcore_axis_name="s")

    @functools.partial(
        pl.kernel,
        mesh=mesh,
        compiler_params=pltpu.CompilerParams(needs_layout_passes=False,
                                             use_tc_tiling_on_sc=False),
        out_type=[
            jax.ShapeDtypeStruct((B * NEG, L), jnp.float32),
            jax.ShapeDtypeStruct((B * NEG, L), jnp.float32),
            jax.ShapeDtypeStruct((B * NEG, L), jnp.float32),
        ],
        scratch_types=[
            pltpu.VMEM((bpw,), jnp.int32),
            pltpu.VMEM((vpw,), jnp.int32),
            pltpu.VMEM((bpw // 8, D), jnp.float32),
            pltpu.VMEM((bpw // 8, D), jnp.float32),
            pltpu.VMEM((vpw // 8, D), jnp.float32),
            pltpu.VMEM((vpw // 8, D), jnp.float32),
            pltpu.VMEM((vpw, L), jnp.float32),
            pltpu.VMEM((vpw, L), jnp.float32),
            pltpu.VMEM((vpw, L), jnp.float32),
            pltpu.SemaphoreType.DMA,
            pltpu.SemaphoreType.DMA,
        ],
    )
    def k(bu_hbm, ni_hbm, tu_hbm, ts_hbm, dot_out, nv2_out, nu2_out,
          uidx_v, nidx_v, u_buf0, u_buf1, v_buf0, v_buf1,
          rdot, rnv2, rnu2, sem0, sem1):
        u_bufs = (u_buf0, u_buf1)
        v_bufs = (v_buf0, v_buf1)
        wid = lax.axis_index("s") * NC + lax.axis_index("c")
        base = wid * bpw
        vbase = wid * vpw
        P = 8
        rp = bpw // P
        vp = vpw // P
        sems = (sem0, sem1)

        pltpu.sync_copy(bu_hbm.at[pl.ds(base, bpw)], uidx_v)
        pltpu.sync_copy(ni_hbm.at[pl.ds(vbase, vpw)], nidx_v)

        def fire(p):
            b = p % 2
            return [
                pltpu.async_copy(tu_hbm.at[uidx_v.at[pl.ds(p * rp, rp)]],
                                 u_bufs[b], sems[b]),
                pltpu.async_copy(ts_hbm.at[nidx_v.at[pl.ds(p * vp, vp)]],
                                 v_bufs[b], sems[b]),
            ]

        pending = {0: fire(0)}
        for p in range(P):
            b = p % 2
            for cp in pending.pop(p):
                cp.wait()
            if p + 1 < P:
                pending[p + 1] = fire(p + 1)

            def row_body(r, carry, b=b, p=p):
                ub, vb = u_bufs[b], v_bufs[b]
                u = [ub[r, pl.ds(kk * L, L)] for kk in range(K)]
                accu = u[0] * u[0]
                for kk in range(1, K):
                    accu = accu + u[kk] * u[kk]
                lr0 = r * NEG
                gr0 = p * vp + r * NEG
                for j in range(NEG):
                    v0 = vb[lr0 + j, pl.ds(0, L)]
                    accd = u[0] * v0
                    accv = v0 * v0
                    for kk in range(1, K):
                        vk = vb[lr0 + j, pl.ds(kk * L, L)]
                        accd = accd + u[kk] * vk
                        accv = accv + vk * vk
                    rdot[gr0 + j, :] = accd
                    rnv2[gr0 + j, :] = accv
                    rnu2[gr0 + j, :] = accu
                return carry

            lax.fori_loop(0, rp, row_body, 0)

        pltpu.sync_copy(rdot, dot_out.at[pl.ds(vbase, vpw)])
        pltpu.sync_copy(rnv2, nv2_out.at[pl.ds(vbase, vpw)])
        pltpu.sync_copy(rnu2, nu2_out.at[pl.ds(vbase, vpw)])

    return k(batch_users, neg_idx_flat, total_user, total_sub)


def _tc_finish(ue, se, dotp, nv2p, nu2p):
    def body(ue_ref, se_ref, dotp_ref, nv2p_ref, nu2p_ref, out_ref):
        u = ue_ref[...]
        s = se_ref[...]
        dot = jnp.sum(u * s, axis=1)
        na = jnp.sqrt(jnp.sum(u * u, axis=1))
        nb = jnp.sqrt(jnp.sum(s * s, axis=1))
        pos_sum = jnp.sum(dot / jnp.maximum(na * nb, EPS))

        ri = lax.broadcasted_iota(jnp.int32, (128, 8), 0)
        ci = lax.broadcasted_iota(jnp.int32, (128, 8), 1)
        m = (ri // 16 == ci).astype(jnp.float32)
        d = jnp.dot(dotp_ref[...], m, preferred_element_type=jnp.float32)
        v2 = jnp.dot(nv2p_ref[...], m, preferred_element_type=jnp.float32)
        u2 = jnp.dot(nu2p_ref[...], m, preferred_element_type=jnp.float32)
        c = d / jnp.maximum(jnp.sqrt(u2) * jnp.sqrt(v2), EPS)
        neg_sum = jnp.sum(c)

        out_ref[0, 0] = 1.0 - pos_sum / B + neg_sum / (NEG * B)

    return pl.pallas_call(
        body,
        out_shape=jax.ShapeDtypeStruct((1, 1), jnp.float32),
        out_specs=pl.BlockSpec(memory_space=pltpu.SMEM),
    )(ue, se, dotp, nv2p, nu2p)


def kernel(user_embeddings, subreddit_embeddings, batch_users,
           batch_subreddits, total_user_embeddings, total_subreddit_embeddings,
           negative_indices):
    del batch_subreddits
    bu = batch_users.astype(jnp.int32)
    ni = negative_indices.astype(jnp.int32).reshape(B * NEG)
    dotp, nv2p, nu2p = _sc_neg_stats(bu, ni, total_user_embeddings,
                                     total_subreddit_embeddings)
    r = B * NEG * 16 // 128
    loss = _tc_finish(user_embeddings, subreddit_embeddings,
                      dotp.reshape(r, 128), nv2p.reshape(r, 128),
                      nu2p.reshape(r, 128))
    return loss[0, 0]
